# baseline (speedup 1.0000x reference)
import functools

import jax
import jax.numpy as jnp
from jax import lax
from jax.experimental import pallas as pl
from jax.experimental.pallas import tpu as pltpu
from jax.experimental.pallas import tpu_sc as plsc

K = 1024
THRESH = 20.0
KWTA = 8
T = 64
RF = 64
LEN = 256
BLK = 256
NBLK = K // BLK

NC = 2
NS = 16
L = 16
ROWS_PER_TILE = T // (NC * NS)


def _tc_kernel(x_ref, w_ref, spike_ref, total_ref, cnt_s, val_s):
    i = pl.program_id(0)
    x = x_ref[...]
    out_blk = jnp.zeros((T, BLK), jnp.float32)
    for r in range(RF):
        out_blk += lax.dot_general(
            x[:, r, :], w_ref[:, r, :], (((1,), (1,)), ((), ())),
            preferred_element_type=jnp.float32)
    pot = jnp.where(out_blk > THRESH, out_blk, 0.0)
    spike = jnp.where(out_blk > THRESH, 1.0, 0.0)
    cnt = jnp.sum(spike, axis=0, keepdims=True)
    first = jnp.clip((T - cnt).astype(jnp.int32), 0, T - 1)
    rows = lax.broadcasted_iota(jnp.int32, (T, BLK), 0)
    vals = jnp.sum(jnp.where(rows == first, pot, 0.0), axis=0,
                   keepdims=True)
    spike_ref[:, pl.ds(i * BLK, BLK)] = spike
    cnt_s[:, pl.ds(i * BLK, BLK)] = cnt
    val_s[:, pl.ds(i * BLK, BLK)] = vals

    @pl.when(i == NBLK - 1)
    def _():
        cnt_all = cnt_s[...]
        val_all = val_s[...]
        v = jnp.max(val_all) * T
        total_ref[...] = cnt_all * (val_all + v)


def _sc_topk_mask(total_hbm, spike_hbm, out_hbm, tot_v, coef_v, rows_v):
    c = lax.axis_index("c")
    s = lax.axis_index("s")
    iota = lax.broadcasted_iota(jnp.int32, (L,), 0)
    nv = K // L

    pltpu.sync_copy(total_hbm.at[0], tot_v)
    zero = jnp.zeros((L,), jnp.float32)
    for j in range(nv):
        coef_v[pl.ds(j * L, L)] = zero

    NG = 8
    GW = K // NG
    gv = GW // L
    def group_max(g0):
        m16 = tot_v[pl.ds(g0, L)]
        for u in range(1, gv):
            m16 = jnp.maximum(m16, tot_v[pl.ds(g0 + u * L, L)])
        return jnp.max(m16)
    sm16 = jnp.zeros((L,), jnp.float32)
    for g in range(NG):
        sm16 = jnp.where(iota == g, group_max(g * GW), sm16)
    sm16 = jnp.where(iota < NG, sm16, -1.0)
    for w in range(KWTA):
        m = jnp.max(sm16)
        g = jnp.min(jnp.where(sm16 == m, iota, NG))
        g0 = g * GW
        idx = jnp.int32(K)
        for u in range(gv):
            tj = tot_v[pl.ds(g0 + u * L, L)]
            idx = jnp.minimum(
                idx, jnp.min(jnp.where(tj == m, g0 + u * L + iota, K)))
        start = idx & jnp.int32(~(L - 1))
        hit = start + iota == idx
        tv = tot_v[pl.ds(start, L)]
        tot_v[pl.ds(start, L)] = jnp.where(hit, 0.0, tv)
        cv = coef_v[pl.ds(start, L)]
        coef_v[pl.ds(start, L)] = jnp.where(hit & (m != 0.0), 1.0, cv)
        sm16 = jnp.where(iota == g, group_max(g0), sm16)

    wid = s * NC + c
    r0 = wid * ROWS_PER_TILE
    pltpu.sync_copy(spike_hbm.at[pl.ds(r0, ROWS_PER_TILE)], rows_v)
    for row in range(ROWS_PER_TILE):
        for j in range(nv):
            sl = pl.ds(j * L, L)
            rows_v[row, sl] = rows_v[row, sl] * coef_v[sl]
    pltpu.sync_copy(rows_v, out_hbm.at[pl.ds(r0, ROWS_PER_TILE)])


_sc_call = functools.partial(
    pl.kernel,
    out_type=jax.ShapeDtypeStruct((T, K), jnp.float32),
    mesh=plsc.VectorSubcoreMesh(core_axis_name="c", subcore_axis_name="s"),
    compiler_params=pltpu.CompilerParams(needs_layout_passes=False, skip_device_barrier=True),
    scratch_types=[
        pltpu.VMEM((K,), jnp.float32),
        pltpu.VMEM((K,), jnp.float32),
        pltpu.VMEM((ROWS_PER_TILE, K), jnp.float32),
    ],
)(_sc_topk_mask)


@jax.jit
def kernel(rec_field, W):
    x = rec_field.reshape(T, RF, LEN)
    w = W.reshape(K, RF, LEN)
    spike, total = pl.pallas_call(
        _tc_kernel,
        grid=(NBLK,),
        in_specs=[
            pl.BlockSpec((T, RF, LEN), lambda i: (0, 0, 0)),
            pl.BlockSpec((BLK, RF, LEN), lambda i: (i, 0, 0)),
        ],
        out_specs=[
            pl.BlockSpec((T, K), lambda i: (0, 0)),
            pl.BlockSpec((1, K), lambda i: (0, 0)),
        ],
        out_shape=[
            jax.ShapeDtypeStruct((T, K), jnp.float32),
            jax.ShapeDtypeStruct((1, K), jnp.float32),
        ],
        scratch_shapes=[
            pltpu.VMEM((1, K), jnp.float32),
            pltpu.VMEM((1, K), jnp.float32),
        ],
    )(x, w)
    out = _sc_call(total, spike)
    return out.reshape(T, K, 1, 1)
